# baseline (speedup 1.0000x reference)
import jax
import jax.numpy as jnp
from jax import lax
from jax.experimental import pallas as pl

_N = 16384
_ROWS = 128
_COLS = 128
_LOG2N = 14
_TOPK_ROWS = 8

_ROTS = ((13, 15, 26, 6), (17, 29, 16, 24))
_PARITY = 0x1BD11BDA
_SIGN = -2147483648


def _rotl(x, d):
    return lax.shift_left(x, jnp.int32(d)) | lax.shift_right_logical(
        x, jnp.int32(32 - d)
    )


def _threefry2x32(k1, k2, x0, x1):
    ks = (k1, k2, k1 ^ k2 ^ jnp.int32(_PARITY))
    x0 = x0 + k1
    x1 = x1 + k2
    for r in range(5):
        for d in _ROTS[r % 2]:
            x0 = x0 + x1
            x1 = _rotl(x1, d)
            x1 = x1 ^ x0
        x0 = x0 + ks[(r + 1) % 3]
        x1 = x1 + ks[(r + 2) % 3] + jnp.int32(r + 1)
    return x0, x1


def _split_pair(k1, k2, counter):
    zero = jnp.int32(0)
    return _threefry2x32(k1, k2, zero, jnp.int32(counter))


def _bits(k1, k2, counts):
    b0, b1 = _threefry2x32(k1, k2, jnp.zeros_like(counts), counts)
    return b0 ^ b1


def _bitonic_sort(keys, arrs, idx, pos):
    for k in range(1, _LOG2N + 1):
        asc = ((pos >> k) & 1) == 0
        for j in range(k - 1, -1, -1):
            s = 1 << j
            if j < 7:
                axis, shift = 1, s
            else:
                axis, shift = 0, s >> 7
            lower = ((pos >> j) & 1) == 0

            def partner(x):
                return jnp.where(
                    lower, jnp.roll(x, -shift, axis=axis), jnp.roll(x, shift, axis=axis)
                )

            pk = partner(keys)
            pi = partner(idx)
            a_lt_p = (keys < pk) | ((keys == pk) & (idx < pi))
            sel_a = (lower == asc) == a_lt_p
            keys = jnp.where(sel_a, keys, pk)
            new_arrs = []
            for x in arrs:
                new_arrs.append(jnp.where(sel_a, x, partner(x)))
            arrs = new_arrs
            idx = jnp.where(sel_a, idx, pi)
    return keys, arrs, idx


def _body(o_ref):
    counts = (
        lax.broadcasted_iota(jnp.int32, (_ROWS, _COLS), 0) * _COLS
        + lax.broadcasted_iota(jnp.int32, (_ROWS, _COLS), 1)
    )

    k0a, k0b = jnp.int32(0), jnp.int32(42)
    key1 = _split_pair(k0a, k0b, 0)
    sub1 = _split_pair(k0a, k0b, 1)
    sub2 = _split_pair(key1[0], key1[1], 1)

    keys1 = _bits(sub1[0], sub1[1], counts) ^ jnp.int32(_SIGN)
    _, (p1,), _ = _bitonic_sort(keys1, (counts,), counts, counts)

    keys2 = _bits(sub2[0], sub2[1], counts) ^ jnp.int32(_SIGN)
    _, (pf,), _ = _bitonic_sort(keys2, (p1,), counts, counts)

    o_ref[...] = pf[:_TOPK_ROWS, :]


def kernel(X):
    del X
    out = pl.pallas_call(
        _body,
        out_shape=jax.ShapeDtypeStruct((_TOPK_ROWS, _COLS), jnp.int32),
    )()
    return out.reshape(_TOPK_ROWS * _COLS).astype(jnp.int64)


# speedup vs baseline: 1.3121x; 1.3121x over previous
import jax
import jax.numpy as jnp
from jax import lax
from jax.experimental import pallas as pl

_N = 16384
_ROWS = 128
_COLS = 128
_LOG2N = 14
_TOPK_ROWS = 8

_ROTS = ((13, 15, 26, 6), (17, 29, 16, 24))
_PARITY = 0x1BD11BDA
_SIGN = -2147483648


def _rotl(x, d):
    return lax.shift_left(x, jnp.int32(d)) | lax.shift_right_logical(
        x, jnp.int32(32 - d)
    )


def _threefry2x32(k1, k2, x0, x1):
    ks = (k1, k2, k1 ^ k2 ^ jnp.int32(_PARITY))
    x0 = x0 + k1
    x1 = x1 + k2
    for r in range(5):
        for d in _ROTS[r % 2]:
            x0 = x0 + x1
            x1 = _rotl(x1, d)
            x1 = x1 ^ x0
        x0 = x0 + ks[(r + 1) % 3]
        x1 = x1 + ks[(r + 2) % 3] + jnp.int32(r + 1)
    return x0, x1


def _split_pair(k1, k2, counter):
    zero = jnp.int32(0)
    return _threefry2x32(k1, k2, zero, jnp.int32(counter))


def _bits(k1, k2, counts):
    b0, b1 = _threefry2x32(k1, k2, jnp.zeros_like(counts), counts)
    return b0 ^ b1


def _pos_bit_clear(bit):
    if bit < 7:
        lane = lax.broadcasted_iota(jnp.int32, (1, _COLS), 1)
        return ((lane >> bit) & 1) == 0
    row = lax.broadcasted_iota(jnp.int32, (_ROWS, 1), 0)
    return ((row >> (bit - 7)) & 1) == 0


def _bitonic_sort(keys, val):
    for k in range(1, _LOG2N + 1):
        asc = _pos_bit_clear(k) if k < _LOG2N else None
        for j in range(k - 1, -1, -1):
            s = 1 << j
            if j < 7:
                axis, shift = 1, s
            else:
                axis, shift = 0, s >> 7
            lower = _pos_bit_clear(j)

            def partner(x):
                return jnp.where(
                    lower, jnp.roll(x, -shift, axis=axis), jnp.roll(x, shift, axis=axis)
                )

            pk = partner(keys)
            pv = partner(val)
            a_lt_p = keys < pk
            take_min = lower if asc is None else lower == asc
            sel_a = take_min == a_lt_p
            keys = jnp.where(sel_a, keys, pk)
            val = jnp.where(sel_a, val, pv)
    return keys, val


def _body(o_ref):
    counts = (
        lax.broadcasted_iota(jnp.int32, (_ROWS, _COLS), 0) * _COLS
        + lax.broadcasted_iota(jnp.int32, (_ROWS, _COLS), 1)
    )

    k0a, k0b = jnp.int32(0), jnp.int32(42)
    key1 = _split_pair(k0a, k0b, 0)
    sub1 = _split_pair(k0a, k0b, 1)
    sub2 = _split_pair(key1[0], key1[1], 1)

    keys1 = _bits(sub1[0], sub1[1], counts) ^ jnp.int32(_SIGN)
    _, p1 = _bitonic_sort(keys1, counts)

    keys2 = _bits(sub2[0], sub2[1], counts) ^ jnp.int32(_SIGN)
    _, pf = _bitonic_sort(keys2, p1)

    o_ref[...] = pf[:_TOPK_ROWS, :]


def kernel(X):
    del X
    out = pl.pallas_call(
        _body,
        out_shape=jax.ShapeDtypeStruct((_TOPK_ROWS, _COLS), jnp.int32),
    )()
    return out.reshape(_TOPK_ROWS * _COLS).astype(jnp.int64)


# speedup vs baseline: 1.4192x; 1.0816x over previous
import jax
import jax.numpy as jnp
from jax import lax
from jax.experimental import pallas as pl

_N = 16384
_ROWS = 128
_COLS = 128
_LOG2N = 14
_TOPK_ROWS = 8

_ROTS = ((13, 15, 26, 6), (17, 29, 16, 24))
_PARITY = 0x1BD11BDA
_SIGN = -2147483648


def _rotl(x, d):
    return lax.shift_left(x, jnp.int32(d)) | lax.shift_right_logical(
        x, jnp.int32(32 - d)
    )


def _threefry2x32(k1, k2, x0, x1):
    ks = (k1, k2, k1 ^ k2 ^ jnp.int32(_PARITY))
    x0 = x0 + k1
    x1 = x1 + k2
    for r in range(5):
        for d in _ROTS[r % 2]:
            x0 = x0 + x1
            x1 = _rotl(x1, d)
            x1 = x1 ^ x0
        x0 = x0 + ks[(r + 1) % 3]
        x1 = x1 + ks[(r + 2) % 3] + jnp.int32(r + 1)
    return x0, x1


def _split_pair(k1, k2, counter):
    zero = jnp.int32(0)
    return _threefry2x32(k1, k2, zero, jnp.int32(counter))


def _bits(k1, k2, counts):
    b0, b1 = _threefry2x32(k1, k2, jnp.zeros_like(counts), counts)
    return b0 ^ b1


def _pos_bit_clear(bit, rows=_ROWS):
    if bit < 7:
        lane = lax.broadcasted_iota(jnp.int32, (1, _COLS), 1)
        return ((lane >> bit) & 1) == 0
    row = lax.broadcasted_iota(jnp.int32, (rows, 1), 0)
    return ((row >> (bit - 7)) & 1) == 0


def _ce_step(keys, val, j, asc):
    s = 1 << j
    if j < 7:
        axis, shift = 1, s
    else:
        axis, shift = 0, s >> 7
    lower = _pos_bit_clear(j, keys.shape[0])

    def partner(x):
        return jnp.where(
            lower, jnp.roll(x, -shift, axis=axis), jnp.roll(x, shift, axis=axis)
        )

    pk = partner(keys)
    pv = partner(val)
    a_lt_p = keys < pk
    take_min = lower if asc is None else lower == asc
    sel_a = take_min == a_lt_p
    return jnp.where(sel_a, keys, pk), jnp.where(sel_a, val, pv)


def _bitonic_sort(keys, val):
    for k in range(1, _LOG2N + 1):
        asc = _pos_bit_clear(k) if k < _LOG2N else None
        for j in range(k - 1, -1, -1):
            keys, val = _ce_step(keys, val, j, asc)
    return keys, val


def _bitonic_top1024(keys, val):
    for k in range(1, 11):
        asc = _pos_bit_clear(k)
        for j in range(k - 1, -1, -1):
            keys, val = _ce_step(keys, val, j, asc)
    while keys.shape[0] > _TOPK_ROWS:
        rows = keys.shape[0]
        pk = jnp.roll(keys, -_TOPK_ROWS, axis=0)
        pv = jnp.roll(val, -_TOPK_ROWS, axis=0)
        a_lt_p = keys < pk
        mk = jnp.where(a_lt_p, keys, pk)
        mv = jnp.where(a_lt_p, val, pv)
        keep = [slice(16 * t, 16 * t + _TOPK_ROWS) for t in range(rows // 16)]
        keys = jnp.concatenate([mk[sl] for sl in keep], axis=0)
        val = jnp.concatenate([mv[sl] for sl in keep], axis=0)
        asc = _pos_bit_clear(10, keys.shape[0])
        for j in range(9, -1, -1):
            keys, val = _ce_step(keys, val, j, asc)
    return keys, val


def _body(o_ref):
    counts = (
        lax.broadcasted_iota(jnp.int32, (_ROWS, _COLS), 0) * _COLS
        + lax.broadcasted_iota(jnp.int32, (_ROWS, _COLS), 1)
    )

    k0a, k0b = jnp.int32(0), jnp.int32(42)
    key1 = _split_pair(k0a, k0b, 0)
    sub1 = _split_pair(k0a, k0b, 1)
    sub2 = _split_pair(key1[0], key1[1], 1)

    keys1 = _bits(sub1[0], sub1[1], counts) ^ jnp.int32(_SIGN)
    _, p1 = _bitonic_sort(keys1, counts)

    keys2 = _bits(sub2[0], sub2[1], counts) ^ jnp.int32(_SIGN)
    _, pf = _bitonic_top1024(keys2, p1)

    o_ref[...] = pf


def kernel(X):
    del X
    out = pl.pallas_call(
        _body,
        out_shape=jax.ShapeDtypeStruct((_TOPK_ROWS, _COLS), jnp.int32),
    )()
    return out.reshape(_TOPK_ROWS * _COLS).astype(jnp.int64)


# speedup vs baseline: 1.7547x; 1.2364x over previous
import jax
import jax.numpy as jnp
from jax import lax
from jax.experimental import pallas as pl

_N = 16384
_ROWS = 128
_COLS = 128
_LOG2N = 14
_TOPK_ROWS = 8

_ROTS = ((13, 15, 26, 6), (17, 29, 16, 24))
_PARITY = 0x1BD11BDA
_SIGN = -2147483648


def _rotl(x, d):
    return lax.shift_left(x, jnp.int32(d)) | lax.shift_right_logical(
        x, jnp.int32(32 - d)
    )


def _threefry2x32(k1, k2, x0, x1):
    ks = (k1, k2, k1 ^ k2 ^ jnp.int32(_PARITY))
    x0 = x0 + k1
    x1 = x1 + k2
    for r in range(5):
        for d in _ROTS[r % 2]:
            x0 = x0 + x1
            x1 = _rotl(x1, d)
            x1 = x1 ^ x0
        x0 = x0 + ks[(r + 1) % 3]
        x1 = x1 + ks[(r + 2) % 3] + jnp.int32(r + 1)
    return x0, x1


def _split_pair(k1, k2, counter):
    zero = jnp.int32(0)
    return _threefry2x32(k1, k2, zero, jnp.int32(counter))


def _bits(k1, k2, counts):
    b0, b1 = _threefry2x32(k1, k2, jnp.zeros_like(counts), counts)
    return b0 ^ b1


def _pos_bit_clear(bit, rows=_ROWS):
    if bit < 7:
        lane = lax.broadcasted_iota(jnp.int32, (1, _COLS), 1)
        return ((lane >> bit) & 1) == 0
    row = lax.broadcasted_iota(jnp.int32, (rows, 1), 0)
    return ((row >> (bit - 7)) & 1) == 0


def _ce_step(keys, val, j, asc):
    s = 1 << j
    if j < 7:
        axis, shift = 1, s
    else:
        axis, shift = 0, s >> 7
    lower = _pos_bit_clear(j, keys.shape[0])
    if axis == 1:
        perm = lax.broadcasted_iota(jnp.int32, (keys.shape[0], _COLS), 1) ^ shift

        def partner(x):
            return jnp.take_along_axis(x, perm, axis=1)

    else:

        def partner(x):
            return jnp.where(
                lower, jnp.roll(x, -shift, axis=0), jnp.roll(x, shift, axis=0)
            )

    pk = partner(keys)
    pv = partner(val)
    a_lt_p = keys < pk
    take_min = lower if asc is None else lower == asc
    sel_a = take_min == a_lt_p
    return jnp.where(sel_a, keys, pk), jnp.where(sel_a, val, pv)


def _bitonic_sort(keys, val):
    for k in range(1, _LOG2N + 1):
        asc = _pos_bit_clear(k) if k < _LOG2N else None
        for j in range(k - 1, -1, -1):
            keys, val = _ce_step(keys, val, j, asc)
    return keys, val


def _bitonic_top1024(keys, val):
    for k in range(1, 11):
        asc = _pos_bit_clear(k)
        for j in range(k - 1, -1, -1):
            keys, val = _ce_step(keys, val, j, asc)
    while keys.shape[0] > _TOPK_ROWS:
        rows = keys.shape[0]
        pk = jnp.roll(keys, -_TOPK_ROWS, axis=0)
        pv = jnp.roll(val, -_TOPK_ROWS, axis=0)
        a_lt_p = keys < pk
        mk = jnp.where(a_lt_p, keys, pk)
        mv = jnp.where(a_lt_p, val, pv)
        keep = [slice(16 * t, 16 * t + _TOPK_ROWS) for t in range(rows // 16)]
        keys = jnp.concatenate([mk[sl] for sl in keep], axis=0)
        val = jnp.concatenate([mv[sl] for sl in keep], axis=0)
        asc = _pos_bit_clear(10, keys.shape[0])
        for j in range(9, -1, -1):
            keys, val = _ce_step(keys, val, j, asc)
    return keys, val


def _body(o_ref):
    counts = (
        lax.broadcasted_iota(jnp.int32, (_ROWS, _COLS), 0) * _COLS
        + lax.broadcasted_iota(jnp.int32, (_ROWS, _COLS), 1)
    )

    k0a, k0b = jnp.int32(0), jnp.int32(42)
    key1 = _split_pair(k0a, k0b, 0)
    sub1 = _split_pair(k0a, k0b, 1)
    sub2 = _split_pair(key1[0], key1[1], 1)

    keys1 = _bits(sub1[0], sub1[1], counts) ^ jnp.int32(_SIGN)
    _, p1 = _bitonic_sort(keys1, counts)

    keys2 = _bits(sub2[0], sub2[1], counts) ^ jnp.int32(_SIGN)
    _, pf = _bitonic_top1024(keys2, p1)

    o_ref[...] = pf


def kernel(X):
    del X
    out = pl.pallas_call(
        _body,
        out_shape=jax.ShapeDtypeStruct((_TOPK_ROWS, _COLS), jnp.int32),
    )()
    return out.reshape(_TOPK_ROWS * _COLS).astype(jnp.int64)


# speedup vs baseline: 2.8111x; 1.6021x over previous
import jax
import jax.numpy as jnp
from jax import lax
from jax.experimental import pallas as pl

_N = 16384
_ROWS = 128
_COLS = 128
_LOG2N = 14
_TOPK_ROWS = 8

_ROTS = ((13, 15, 26, 6), (17, 29, 16, 24))
_PARITY = 0x1BD11BDA
_SIGN = -2147483648


def _rotl(x, d):
    return lax.shift_left(x, jnp.int32(d)) | lax.shift_right_logical(
        x, jnp.int32(32 - d)
    )


def _threefry2x32(k1, k2, x0, x1):
    ks = (k1, k2, k1 ^ k2 ^ jnp.int32(_PARITY))
    x0 = x0 + k1
    x1 = x1 + k2
    for r in range(5):
        for d in _ROTS[r % 2]:
            x0 = x0 + x1
            x1 = _rotl(x1, d)
            x1 = x1 ^ x0
        x0 = x0 + ks[(r + 1) % 3]
        x1 = x1 + ks[(r + 2) % 3] + jnp.int32(r + 1)
    return x0, x1


def _split_pair(k1, k2, counter):
    zero = jnp.int32(0)
    return _threefry2x32(k1, k2, zero, jnp.int32(counter))


def _bits(k1, k2, counts):
    b0, b1 = _threefry2x32(k1, k2, jnp.zeros_like(counts), counts)
    return b0 ^ b1


def _pos_bit_clear(bit, rows=_ROWS):
    if bit < 7:
        lane = lax.broadcasted_iota(jnp.int32, (1, _COLS), 1)
        return ((lane >> bit) & 1) == 0
    row = lax.broadcasted_iota(jnp.int32, (rows, 1), 0)
    return ((row >> (bit - 7)) & 1) == 0


def _ce_step(keys, val, j, asc):
    s = 1 << j
    if j < 7:
        axis, shift = 1, s
    else:
        axis, shift = 0, s >> 7
    lower = _pos_bit_clear(j, keys.shape[0])
    if axis == 1:
        perm = lax.broadcasted_iota(jnp.int32, (keys.shape[0], _COLS), 1) ^ shift

        def partner(x):
            return jnp.take_along_axis(x, perm, axis=1)

    else:

        def partner(x):
            return jnp.where(
                lower, jnp.roll(x, -shift, axis=0), jnp.roll(x, shift, axis=0)
            )

    pk = partner(keys)
    pv = partner(val)
    a_lt_p = keys < pk
    take_min = lower if asc is None else lower == asc
    sel_a = take_min == a_lt_p
    return jnp.where(sel_a, keys, pk), jnp.where(sel_a, val, pv)


def _bitonic_sort(keys, val):
    for k in range(1, _LOG2N + 1):
        asc = _pos_bit_clear(k) if k < _LOG2N else None
        for j in range(k - 1, -1, -1):
            keys, val = _ce_step(keys, val, j, asc)
    return keys, val


def _bitonic_top1024(keys, val):
    for k in range(1, 11):
        asc = _pos_bit_clear(k)
        for j in range(k - 1, -1, -1):
            keys, val = _ce_step(keys, val, j, asc)
    while keys.shape[0] > _TOPK_ROWS:
        rows = keys.shape[0]
        pk = jnp.roll(keys, -_TOPK_ROWS, axis=0)
        pv = jnp.roll(val, -_TOPK_ROWS, axis=0)
        a_lt_p = keys < pk
        mk = jnp.where(a_lt_p, keys, pk)
        mv = jnp.where(a_lt_p, val, pv)
        keep = [slice(16 * t, 16 * t + _TOPK_ROWS) for t in range(rows // 16)]
        keys = jnp.concatenate([mk[sl] for sl in keep], axis=0)
        val = jnp.concatenate([mv[sl] for sl in keep], axis=0)
        asc = _pos_bit_clear(10, keys.shape[0])
        for j in range(9, -1, -1):
            keys, val = _ce_step(keys, val, j, asc)
    return keys, val


def _body(o_ref):
    counts = (
        lax.broadcasted_iota(jnp.int32, (_ROWS, _COLS), 0) * _COLS
        + lax.broadcasted_iota(jnp.int32, (_ROWS, _COLS), 1)
    )

    k0a, k0b = jnp.int32(0), jnp.int32(42)
    key1 = _split_pair(k0a, k0b, 0)
    sub1 = _split_pair(k0a, k0b, 1)
    sub2 = _split_pair(key1[0], key1[1], 1)

    keys1 = _bits(sub1[0], sub1[1], counts) ^ jnp.int32(_SIGN)
    _, p1 = _bitonic_sort(keys1, counts)

    keys2 = _bits(sub2[0], sub2[1], counts) ^ jnp.int32(_SIGN)
    _, g = _bitonic_top1024(keys2, counts)

    row_g = g >> 7
    lane_g = g & 127
    pf = jnp.zeros((_TOPK_ROWS, _COLS), jnp.int32)
    for r in range(_ROWS):
        src = jnp.broadcast_to(p1[r : r + 1, :], (_TOPK_ROWS, _COLS))
        picked = jnp.take_along_axis(src, lane_g, axis=1)
        pf = jnp.where(row_g == r, picked, pf)

    o_ref[...] = pf


def kernel(X):
    del X
    out = pl.pallas_call(
        _body,
        out_shape=jax.ShapeDtypeStruct((_TOPK_ROWS, _COLS), jnp.int32),
    )()
    return out.reshape(_TOPK_ROWS * _COLS).astype(jnp.int64)
